# Initial kernel scaffold; baseline (speedup 1.0000x reference)
import jax
import jax.numpy as jnp
from jax.experimental import pallas as pl


def kernel(x, edge_index, W1, b1, W2, b2):
    raise NotImplementedError("write your pallas kernel here")


# speedup vs baseline: 14.5176x; 14.5176x over previous
import functools

import jax
import jax.numpy as jnp
from jax import lax
from jax.experimental import pallas as pl
from jax.experimental.pallas import tpu as pltpu
from jax.experimental.pallas import tpu_sc as plsc

N = 10000
E = 320000
D_IN = 128
D_HID = 256
D_OUT = 128

NPAD = 10240
NC, NS, L = 2, 16, 16
CH = 128
EPAD = 323584
CPT = EPAD // (NS * CH)
CPT_DEG = EPAD // (NC * NS * CH)
PAD_COL = N + 100
RPT = NPAD // NS

_mesh = plsc.VectorSubcoreMesh(core_axis_name="c", subcore_axis_name="s")


@functools.partial(
    pl.kernel,
    out_type=jax.ShapeDtypeStruct((NC * NPAD,), jnp.float32),
    mesh=_mesh,
    scratch_types=[
        pltpu.VMEM((CPT_DEG, CH), jnp.int32),
        pltpu.VMEM((CH,), jnp.float32),
        pltpu.VMEM((RPT,), jnp.float32),
        pltpu.VMEM_SHARED((NPAD,), jnp.float32),
    ],
    compiler_params=pltpu.CompilerParams(use_tc_tiling_on_sc=False),
)
def _deg_kernel(col_h, deg_h, icol, ones, zb, acc):
    c = lax.axis_index("c")
    s = lax.axis_index("s")

    def zf(i, _):
        zb[pl.ds(i * L, L)] = jnp.zeros((L,), jnp.float32)
        return 0

    lax.fori_loop(0, RPT // L, zf, 0)

    def of(i, _):
        ones[pl.ds(i * L, L)] = jnp.ones((L,), jnp.float32)
        return 0

    lax.fori_loop(0, CH // L, of, 0)

    pltpu.sync_copy(zb, acc.at[pl.ds(s * RPT, RPT)])
    plsc.subcore_barrier()

    t = c * NS + s
    pltpu.sync_copy(col_h.at[t], icol)

    def body(j, _):
        pltpu.sync_copy(ones, acc.at[icol.at[j]], add=True)
        return 0

    lax.fori_loop(0, CPT_DEG, body, 0)
    plsc.subcore_barrier()
    pltpu.sync_copy(acc.at[pl.ds(s * RPT, RPT)],
                    deg_h.at[pl.ds(c * NPAD + s * RPT, RPT)])


def _make_scatter(dh):
    @functools.partial(
        pl.kernel,
        out_type=jax.ShapeDtypeStruct((NC * NPAD, dh), jnp.float32),
        mesh=_mesh,
        scratch_types=[
            pltpu.VMEM((CPT, CH), jnp.int32),
            pltpu.VMEM((CPT, CH), jnp.int32),
            pltpu.VMEM((2, CH, dh), jnp.float32),
            pltpu.VMEM((64, dh), jnp.float32),
            pltpu.VMEM_SHARED((NPAD, dh), jnp.float32),
            pltpu.SemaphoreType.DMA,
            pltpu.SemaphoreType.DMA,
        ],
        compiler_params=pltpu.CompilerParams(use_tc_tiling_on_sc=False),
    )
    def _scat(zt_h, rowa_h, rowb_h, col_h, agg_h,
              irow, icol, gbuf, zb, acc, sem0, sem1):
        c = lax.axis_index("c")
        s = lax.axis_index("s")

        kk = dh // L

        def zf(i, _):
            zb[i // kk, pl.ds((i % kk) * L, L)] = jnp.zeros((L,), jnp.float32)
            return 0

        lax.fori_loop(0, 64 * kk, zf, 0)
        for k in range(RPT // 64):
            pltpu.sync_copy(zb, acc.at[pl.ds(s * RPT + k * 64, 64)])
        plsc.subcore_barrier()

        pltpu.sync_copy(col_h.at[s], icol)

        @pl.when(c == 0)
        def _():
            pltpu.sync_copy(rowa_h.at[s], irow)

        @pl.when(c != 0)
        def _():
            pltpu.sync_copy(rowb_h.at[s], irow)

        pltpu.async_copy(zt_h.at[irow.at[0]], gbuf.at[0], sem0)
        pltpu.async_copy(zt_h.at[irow.at[1]], gbuf.at[1], sem1)

        def body(it, _):
            for b in range(2):
                jj = it * 2 + b
                sem = (sem0, sem1)[b]
                pltpu.make_async_copy(zt_h.at[irow.at[jj]], gbuf.at[b], sem).wait()
                pltpu.sync_copy(gbuf.at[b], acc.at[icol.at[jj]], add=True)

                @pl.when(jj + 2 < CPT)
                def _():
                    pltpu.async_copy(zt_h.at[irow.at[jj + 2]], gbuf.at[b], sem)

            return 0

        lax.fori_loop(0, CPT // 2, body, 0)
        plsc.subcore_barrier()
        pltpu.sync_copy(acc.at[pl.ds(s * RPT, RPT)],
                        agg_h.at[pl.ds(c * NPAD + s * RPT, RPT)])

    return _scat


_scatter64 = _make_scatter(64)


MB = 1024
G = NPAD // MB


def _tc1_body(x_ref, d0_ref, d1_ref, w_ref, z_ref, dv_ref):
    deg = d0_ref[...] + d1_ref[...] + 1.0
    dv = lax.rsqrt(deg)
    xs = x_ref[...] * dv
    z = jnp.dot(xs, w_ref[...], preferred_element_type=jnp.float32)
    for q in range(4):
        z_ref[q] = z[:, q * 64:(q + 1) * 64]
    dv_ref[...] = dv


def _tc1(x_pad, d0, d1, w1):
    return pl.pallas_call(
        _tc1_body,
        grid=(G,),
        in_specs=[
            pl.BlockSpec((MB, D_IN), lambda i: (i, 0)),
            pl.BlockSpec((MB, 1), lambda i: (i, 0)),
            pl.BlockSpec((MB, 1), lambda i: (i, 0)),
            pl.BlockSpec((D_IN, D_HID), lambda i: (0, 0)),
        ],
        out_specs=[
            pl.BlockSpec((4, MB, 64), lambda i: (0, i, 0)),
            pl.BlockSpec((MB, 1), lambda i: (i, 0)),
        ],
        out_shape=[
            jax.ShapeDtypeStruct((4, NPAD, 64), jnp.float32),
            jax.ShapeDtypeStruct((NPAD, 1), jnp.float32),
        ],
    )(x_pad, d0, d1, w1)


def _tc2_body(a01_ref, a23_ref, z_ref, dv_ref, b1_ref, w_ref, o_ref):
    dv = dv_ref[...]
    ts = []
    for q in range(4):
        a = a01_ref[q] if q < 2 else a23_ref[q - 2]
        ts.append(jnp.maximum((a + z_ref[q]) * dv + b1_ref[q, :][None, :], 0.0))
    hcat = jnp.concatenate(ts, axis=1)
    u = jnp.dot(hcat, w_ref[...], preferred_element_type=jnp.float32)
    zz = u * dv
    o_ref[0] = zz[:, : D_OUT // 2]
    o_ref[1] = zz[:, D_OUT // 2:]


def _tc2(agg_a, agg_b, z1s, dinv, b1r, w2):
    return pl.pallas_call(
        _tc2_body,
        grid=(G,),
        in_specs=[
            pl.BlockSpec((2, MB, 64), lambda i: (0, i, 0)),
            pl.BlockSpec((2, MB, 64), lambda i: (0, i, 0)),
            pl.BlockSpec((4, MB, 64), lambda i: (0, i, 0)),
            pl.BlockSpec((MB, 1), lambda i: (i, 0)),
            pl.BlockSpec((4, 64), lambda i: (0, 0)),
            pl.BlockSpec((D_HID, D_OUT), lambda i: (0, 0)),
        ],
        out_specs=pl.BlockSpec((2, MB, D_OUT // 2), lambda i: (0, i, 0)),
        out_shape=jax.ShapeDtypeStruct((2, NPAD, D_OUT // 2), jnp.float32),
    )(agg_a, agg_b, z1s, dinv, b1r, w2)


def _tc3_body(a_ref, z_ref, dv_ref, b2_ref, y_ref):
    dv = dv_ref[...]
    y0 = (a_ref[0] + z_ref[0]) * dv + b2_ref[0, :][None, :]
    y1 = (a_ref[1] + z_ref[1]) * dv + b2_ref[1, :][None, :]
    y_ref[...] = jnp.concatenate([y0, y1], axis=1)


def _tc3(agg2, z2s, dinv, b2r):
    return pl.pallas_call(
        _tc3_body,
        grid=(G,),
        in_specs=[
            pl.BlockSpec((2, MB, D_OUT // 2), lambda i: (0, i, 0)),
            pl.BlockSpec((2, MB, D_OUT // 2), lambda i: (0, i, 0)),
            pl.BlockSpec((MB, 1), lambda i: (i, 0)),
            pl.BlockSpec((2, D_OUT // 2), lambda i: (0, 0)),
        ],
        out_specs=pl.BlockSpec((MB, D_OUT), lambda i: (i, 0)),
        out_shape=jax.ShapeDtypeStruct((NPAD, D_OUT), jnp.float32),
    )(agg2, z2s, dinv, b2r)


def kernel(x, edge_index, W1, b1, W2, b2):
    row = edge_index[0].astype(jnp.int32)
    col = edge_index[1].astype(jnp.int32)
    npad_e = EPAD - E
    row_p = jnp.concatenate([row, jnp.zeros((npad_e,), jnp.int32)])
    col_p = jnp.concatenate([col, jnp.full((npad_e,), PAD_COL, jnp.int32)])
    rowb_p = row_p + NPAD

    col_deg = col_p.reshape(NC * NS, CPT_DEG, CH)
    row_sc = row_p.reshape(NS, CPT, CH)
    rowb_sc = rowb_p.reshape(NS, CPT, CH)
    col_sc = col_p.reshape(NS, CPT, CH)

    x_pad = jnp.pad(x, ((0, NPAD - N), (0, 0)))
    b1r = b1.reshape(4, 64)
    b2r = b2.reshape(2, D_OUT // 2)

    degp = _deg_kernel(col_deg)
    d0 = degp[:NPAD].reshape(NPAD, 1)
    d1 = degp[NPAD:].reshape(NPAD, 1)

    z1s, dinv = _tc1(x_pad, d0, d1, W1)
    agg_a = _scatter64(z1s[:2].reshape(NC * NPAD, 64),
                       row_sc, rowb_sc, col_sc)
    agg_b = _scatter64(z1s[2:].reshape(NC * NPAD, 64),
                       row_sc, rowb_sc, col_sc)
    z2s = _tc2(agg_a.reshape(NC, NPAD, 64), agg_b.reshape(NC, NPAD, 64),
               z1s, dinv, b1r, W2)
    agg2 = _scatter64(z2s.reshape(NC * NPAD, D_OUT // 2),
                      row_sc, rowb_sc, col_sc)
    y = _tc3(agg2.reshape(NC, NPAD, D_OUT // 2), z2s, dinv, b2r)
    return y[:N]
